# Initial kernel scaffold; baseline (speedup 1.0000x reference)
import jax
import jax.numpy as jnp
from jax.experimental import pallas as pl


def kernel(in_feat, edge1, edge2, edge3, edge4, edge5, params, edge_index1, edge_index3, edge_index4, edge_index5):
    raise NotImplementedError("write your pallas kernel here")


# speedup vs baseline: 1.0684x; 1.0684x over previous
import functools

import jax
import jax.numpy as jnp
from jax import lax
from jax.experimental import pallas as pl
from jax.experimental.pallas import tpu as pltpu
from jax.experimental.pallas import tpu_sc as plsc

F32 = jnp.float32

RES = 64
N1 = 6 * RES * RES
N3 = 6 * (RES // 4) ** 2
N4 = 6 * (RES // 8) ** 2
N5 = 6 * (RES // 16) ** 2
E1, E3, E4, E5 = 4 * N1, 4 * N3, 4 * N4, 4 * N5
EH = 32


def _mm_kernel(x_ref, w_ref, b_ref, o_ref):
    o_ref[...] = (jnp.dot(x_ref[...], w_ref[...], preferred_element_type=F32)
                  + b_ref[...])


def _mm(x, w, b):
    M, K = x.shape
    N = w.shape[1]
    return pl.pallas_call(
        _mm_kernel,
        out_shape=jax.ShapeDtypeStruct((M, N), F32),
    )(x, w, b.reshape(1, N))


def _pool_kernel(p_ref, x_ref, o_ref):
    o_ref[0] = lax.dot_general(p_ref[...], x_ref[0],
                               (((0,), (0,)), ((), ())),
                               preferred_element_type=F32)


def _pool(x, pmat):
    _, R, C = x.shape
    r = pmat.shape[1]
    return pl.pallas_call(
        _pool_kernel,
        grid=(6,),
        in_specs=[pl.BlockSpec((R, r), lambda i: (0, 0)),
                  pl.BlockSpec((1, R, C), lambda i: (i, 0, 0))],
        out_specs=pl.BlockSpec((1, r, C), lambda i: (i, 0, 0)),
        out_shape=jax.ShapeDtypeStruct((6, r, C), F32),
    )(pmat, x)


def _proj2_kernel(x_ref, w1_ref, b1_ref, w2_ref, b2_ref, o_ref):
    h = jnp.maximum(jnp.dot(x_ref[...], w1_ref[...],
                            preferred_element_type=F32) + b1_ref[...], 0.0)
    o_ref[...] = (jnp.dot(h, w2_ref[...], preferred_element_type=F32)
                  + b2_ref[...])


def _proj2(x, w1t, b1, w2t, b2, block_m=None):
    M, K = x.shape
    H = w1t.shape[1]
    O = w2t.shape[1]
    if block_m is None:
        block_m = M
    return pl.pallas_call(
        _proj2_kernel,
        grid=(M // block_m,),
        in_specs=[pl.BlockSpec((block_m, K), lambda i: (i, 0)),
                  pl.BlockSpec((K, H), lambda i: (0, 0)),
                  pl.BlockSpec((1, H), lambda i: (0, 0)),
                  pl.BlockSpec((H, O), lambda i: (0, 0)),
                  pl.BlockSpec((1, O), lambda i: (0, 0))],
        out_specs=pl.BlockSpec((block_m, O), lambda i: (i, 0)),
        out_shape=jax.ShapeDtypeStruct((M, O), F32),
    )(x, w1t, b1.reshape(1, H), w2t, b2.reshape(1, O))


def _level_kernel(x_ref, ef_ref, src_ref, dst_ref,
                  pn1t, pn1b, pn2t, pn2b, en1t, en1b, w3, bm, convb,
                  wiht, bih, whht, bhh, out_ref, ewh_ref, agg_ref,
                  *, N, E, H, CE):
    h = jnp.maximum(jnp.dot(x_ref[...], pn1t[...],
                            preferred_element_type=F32) + pn1b[...], 0.0)
    h = jnp.dot(h, pn2t[...], preferred_element_type=F32) + pn2b[...]
    hidden = h
    ewh_ref[...] = jnp.maximum(
        jnp.dot(ef_ref[...], en1t[...], preferred_element_type=F32)
        + en1b[...], 0.0)
    nch = E // CE

    def deg_body(c, deg):
        d = dst_ref[pl.ds(pl.multiple_of(c * CE, CE), CE), :]
        ohd = (lax.broadcasted_iota(jnp.int32, (CE, N), 1) == d).astype(F32)
        return deg + lax.dot_general(ohd, jnp.ones((CE, 1), F32),
                                     (((0,), (0,)), ((), ())),
                                     preferred_element_type=F32)

    deg = lax.fori_loop(0, nch, deg_body, jnp.zeros((N, 1), F32))
    dinv = 1.0 / jnp.maximum(deg, 1.0)

    for _ in range(3):
        node = hidden
        agg_ref[...] = jnp.zeros((N, H), F32)

        def chunk_body(c, _):
            o = pl.multiple_of(c * CE, CE)
            s = src_ref[pl.ds(o, CE), :]
            d = dst_ref[pl.ds(o, CE), :]
            iot = lax.broadcasted_iota(jnp.int32, (CE, N), 1)
            ohs = (iot == s).astype(F32)
            ohd = (iot == d).astype(F32)
            xs = jnp.dot(ohs, node, preferred_element_type=F32)
            ec = ewh_ref[pl.ds(o, CE), :]
            z = (xs[:, :, None] * ec[:, None, :]).reshape(CE, H * EH)
            msg = (jnp.dot(z, w3[...], preferred_element_type=F32)
                   + jnp.dot(xs, bm[...], preferred_element_type=F32))
            agg_ref[...] += lax.dot_general(ohd, msg, (((0,), (0,)), ((), ())),
                                            preferred_element_type=F32)
            return 0

        lax.fori_loop(0, nch, chunk_body, 0)
        m = jnp.maximum(agg_ref[...] * dinv + convb[...], 0.0)
        gi = jnp.dot(m, wiht[...], preferred_element_type=F32) + bih[...]
        gh = jnp.dot(hidden, whht[...], preferred_element_type=F32) + bhh[...]
        r = jax.nn.sigmoid(gi[:, :H] + gh[:, :H])
        zg = jax.nn.sigmoid(gi[:, H:2 * H] + gh[:, H:2 * H])
        n = jnp.tanh(gi[:, 2 * H:] + r * gh[:, 2 * H:])
        hidden = (1.0 - zg) * n + zg * hidden
    out_ref[...] = hidden


def _w3_bm(p, H):
    w3 = p['en2_w'].reshape(H, H, EH).transpose(0, 2, 1).reshape(H * EH, H)
    bm = p['en2_b'].reshape(H, H)
    return w3, bm


def _level(p, x, ef, ei, N, H, E):
    CE = min(E, 512)
    w3, bm = _w3_bm(p, H)
    kfn = functools.partial(_level_kernel, N=N, E=E, H=H, CE=CE)
    return pl.pallas_call(
        kfn,
        out_shape=jax.ShapeDtypeStruct((N, H), F32),
        scratch_shapes=[pltpu.VMEM((E, EH), F32),
                        pltpu.VMEM((N, H), F32)],
    )(x, ef, ei[0].reshape(E, 1), ei[1].reshape(E, 1),
      p['pn1_w'].T, p['pn1_b'].reshape(1, H),
      p['pn2_w'].T, p['pn2_b'].reshape(1, H),
      p['en1_w'].T, p['en1_b'].reshape(1, EH),
      w3, bm, p['conv_b'].reshape(1, H),
      p['gru_wih'].T, p['gru_bih'].reshape(1, 3 * H),
      p['gru_whh'].T, p['gru_bhh'].reshape(1, 3 * H))


_SC_MESH = plsc.VectorSubcoreMesh(core_axis_name="c", subcore_axis_name="s")
_NW = 32
_LW = 128


def _sc_gather(table, idx2d, E):
    bpw = E // _NW
    nch = bpw // 1024

    @functools.partial(
        pl.kernel,
        out_type=jax.ShapeDtypeStruct((E, _LW), F32),
        mesh=_SC_MESH,
        scratch_types=[pltpu.VMEM((8, 128), jnp.int32),
                       pltpu.VMEM((512, _LW), F32),
                       pltpu.SemaphoreType.DMA],
    )
    def k(table_hbm, idx_hbm, out_hbm, idx_v, rows_v, sem):
        wid = lax.axis_index("c") * 16 + lax.axis_index("s")
        base = wid * bpw

        @pl.loop(0, nch)
        def _chunk(ci):
            e0 = pl.multiple_of(base + ci * 1024, 1024)
            pltpu.sync_copy(
                idx_hbm.at[pl.ds(pl.multiple_of(e0 // 128, 8), 8)], idx_v)
            for h in range(2):
                hs = []
                for j in range(4):
                    hs.append(pltpu.async_copy(
                        table_hbm.at[idx_v.at[h * 4 + j]],
                        rows_v.at[pl.ds(j * 128, 128)], sem))
                for hdl in hs:
                    hdl.wait()
                pltpu.sync_copy(
                    rows_v,
                    out_hbm.at[pl.ds(pl.multiple_of(e0 + h * 512, 512), 512)])

    return k(table, idx2d)


def _sc_scatter_add(msg, dst2, zeros, N, E):
    half = N // 2
    bps = E // 16
    nch = bps // 1024

    @functools.partial(
        pl.kernel,
        out_type=jax.ShapeDtypeStruct((N, _LW), F32),
        mesh=_SC_MESH,
        scratch_types=[pltpu.VMEM((8, 128), jnp.int32),
                       pltpu.VMEM((128, _LW), F32),
                       pltpu.VMEM_SHARED((16 * 776, _LW), F32),
                       pltpu.SemaphoreType.DMA],
    )
    def k(msg_hbm, idx_hbm, z_hbm, out_hbm, idx_v, msg_v, acc, sem):
        c = lax.axis_index("c")
        s = lax.axis_index("s")
        pltpu.sync_copy(z_hbm, acc.at[pl.ds(pl.multiple_of(s * 776, 8), 776)])
        plsc.subcore_barrier()
        base = s * bps

        @pl.loop(0, nch)
        def _chunk(ci):
            e0 = pl.multiple_of(base + ci * 1024, 1024)
            pltpu.sync_copy(
                idx_hbm.at[c].at[pl.ds(pl.multiple_of(e0 // 128, 8), 8)],
                idx_v)
            for h in range(8):
                pltpu.sync_copy(
                    msg_hbm.at[pl.ds(pl.multiple_of(e0 + h * 128, 128), 128)],
                    msg_v)
                pltpu.async_copy(msg_v, acc.at[idx_v.at[h]], sem,
                                 add=True).wait()
        plsc.subcore_barrier()
        r0 = pl.multiple_of(s * (half // 16), 8)
        w0 = pl.multiple_of(c * half + s * (half // 16), 8)
        pltpu.sync_copy(acc.at[pl.ds(r0, half // 16)],
                        out_hbm.at[pl.ds(w0, half // 16)])

    return k(msg, dst2, zeros)


def _msg_kernel(xs_ref, sm_ref, ef_ref, en1t, en1b, w3, bm, o_ref,
                *, H, with_ones):
    x128 = xs_ref[...]
    sm = sm_ref[...]
    EB = x128.shape[0]
    xs = jnp.zeros((EB, H), F32)
    for kk in range(4):
        sel = (sm == kk).astype(F32)
        xs = xs + sel * x128[:, kk * H:(kk + 1) * H]
    ewh = jnp.maximum(jnp.dot(ef_ref[...], en1t[...],
                              preferred_element_type=F32) + en1b[...], 0.0)
    z = (xs[:, :, None] * ewh[:, None, :]).reshape(EB, H * EH)
    msg = (jnp.dot(z, w3[...], preferred_element_type=F32)
           + jnp.dot(xs, bm[...], preferred_element_type=F32))
    deg = jnp.ones if with_ones else jnp.zeros
    o_ref[...] = jnp.concatenate(
        [msg, deg((EB, 16), F32), jnp.zeros((EB, 128 - H - 16), F32)], axis=1)


def _msg(xs, sm, ef, en1t, en1b, w3, bm, H, with_ones):
    E = xs.shape[0]
    EB = 2048
    kfn = functools.partial(_msg_kernel, H=H, with_ones=with_ones)
    return pl.pallas_call(
        kfn,
        grid=(E // EB,),
        in_specs=[pl.BlockSpec((EB, 128), lambda i: (i, 0)),
                  pl.BlockSpec((EB, 1), lambda i: (i, 0)),
                  pl.BlockSpec((EB, 4), lambda i: (i, 0)),
                  pl.BlockSpec((4, EH), lambda i: (0, 0)),
                  pl.BlockSpec((1, EH), lambda i: (0, 0)),
                  pl.BlockSpec((H * EH, H), lambda i: (0, 0)),
                  pl.BlockSpec((H, H), lambda i: (0, 0))],
        out_specs=pl.BlockSpec((EB, 128), lambda i: (i, 0)),
        out_shape=jax.ShapeDtypeStruct((E, 128), F32),
    )(xs, sm, ef, en1t, en1b.reshape(1, EH), w3, bm)


def _gru1_kernel(agg_ref, hid_ref, convb, wiht, bih, whht, bhh,
                 out_ref, dinv_ref, *, H):
    agg = agg_ref[:, :H]
    deg = agg_ref[:, H:H + 1]
    dinv = 1.0 / jnp.maximum(deg, 1.0)
    dinv_ref[...] = dinv
    _gru_core(agg, dinv, hid_ref, convb, wiht, bih, whht, bhh, out_ref, H)


def _gru23_kernel(agg_ref, dinv_ref, hid_ref, convb, wiht, bih, whht, bhh,
                  out_ref, *, H):
    _gru_core(agg_ref[:, :H], dinv_ref[...], hid_ref, convb, wiht, bih,
              whht, bhh, out_ref, H)


def _gru_core(agg, dinv, hid_ref, convb, wiht, bih, whht, bhh, out_ref, H):
    hidden = hid_ref[...]
    m = jnp.maximum(agg * dinv + convb[...], 0.0)
    gi = jnp.dot(m, wiht[...], preferred_element_type=F32) + bih[...]
    gh = jnp.dot(hidden, whht[...], preferred_element_type=F32) + bhh[...]
    r = jax.nn.sigmoid(gi[:, :H] + gh[:, :H])
    zg = jax.nn.sigmoid(gi[:, H:2 * H] + gh[:, H:2 * H])
    n = jnp.tanh(gi[:, 2 * H:] + r * gh[:, 2 * H:])
    out_ref[...] = (1.0 - zg) * n + zg * hidden


def _gru1(agg, hidden, p, N, H):
    NB = 8192
    kfn = functools.partial(_gru1_kernel, H=H)
    return pl.pallas_call(
        kfn,
        grid=(N // NB,),
        in_specs=[pl.BlockSpec((NB, 128), lambda i: (i, 0)),
                  pl.BlockSpec((NB, H), lambda i: (i, 0)),
                  pl.BlockSpec((1, H), lambda i: (0, 0)),
                  pl.BlockSpec((H, 3 * H), lambda i: (0, 0)),
                  pl.BlockSpec((1, 3 * H), lambda i: (0, 0)),
                  pl.BlockSpec((H, 3 * H), lambda i: (0, 0)),
                  pl.BlockSpec((1, 3 * H), lambda i: (0, 0))],
        out_specs=[pl.BlockSpec((NB, H), lambda i: (i, 0)),
                   pl.BlockSpec((NB, 1), lambda i: (i, 0))],
        out_shape=[jax.ShapeDtypeStruct((N, H), F32),
                   jax.ShapeDtypeStruct((N, 1), F32)],
    )(agg, hidden, p['conv_b'].reshape(1, H),
      p['gru_wih'].T, p['gru_bih'].reshape(1, 3 * H),
      p['gru_whh'].T, p['gru_bhh'].reshape(1, 3 * H))


def _gru23(agg, dinv, hidden, p, N, H):
    NB = 8192
    kfn = functools.partial(_gru23_kernel, H=H)
    return pl.pallas_call(
        kfn,
        grid=(N // NB,),
        in_specs=[pl.BlockSpec((NB, 128), lambda i: (i, 0)),
                  pl.BlockSpec((NB, 1), lambda i: (i, 0)),
                  pl.BlockSpec((NB, H), lambda i: (i, 0)),
                  pl.BlockSpec((1, H), lambda i: (0, 0)),
                  pl.BlockSpec((H, 3 * H), lambda i: (0, 0)),
                  pl.BlockSpec((1, 3 * H), lambda i: (0, 0)),
                  pl.BlockSpec((H, 3 * H), lambda i: (0, 0)),
                  pl.BlockSpec((1, 3 * H), lambda i: (0, 0))],
        out_specs=pl.BlockSpec((NB, H), lambda i: (i, 0)),
        out_shape=jax.ShapeDtypeStruct((N, H), F32),
    )(agg, dinv, hidden, p['conv_b'].reshape(1, H),
      p['gru_wih'].T, p['gru_bih'].reshape(1, 3 * H),
      p['gru_whh'].T, p['gru_bhh'].reshape(1, 3 * H))


def _level1(p, x, ef, ei):
    H = 32
    N, E = N1, E1
    half = N // 2
    w3, bm = _w3_bm(p, H)
    en1t = p['en1_w'].T
    src, dst = ei[0], ei[1]
    srcdiv = (src // 4).reshape(E // 128, 128)
    srcmod = (src % 4).reshape(E, 1)
    dst_lo = jnp.where(dst < half, dst, half)
    dst_hi = jnp.where(dst >= half, dst - half, half)
    dst2 = jnp.stack([dst_lo, dst_hi]).reshape(2, E // 128, 128)
    zeros = jnp.zeros((776, 128), F32)

    hidden = _proj2(x, p['pn1_w'].T, p['pn1_b'], p['pn2_w'].T, p['pn2_b'],
                    block_m=4096)
    dinv = None
    for step in range(3):
        xs = _sc_gather(hidden.reshape(N // 4, 128), srcdiv, E)
        first = step == 0
        msg = _msg(xs, srcmod, ef, en1t, p['en1_b'], w3, bm, H,
                   with_ones=first)
        agg = _sc_scatter_add(msg, dst2, zeros, N, E)
        if first:
            hidden, dinv = _gru1(agg, hidden, p, N, H)
        else:
            hidden = _gru23(agg, dinv, hidden, p, N, H)
    return _proj2(hidden, p['dec1_w'].T, p['dec1_b'],
                  p['dec2_w'].T, p['dec2_b'], block_m=4096)


def _pool_mat(side, p):
    c = jnp.arange(side * side)
    o = jnp.arange((side // p) ** 2)
    oy = o[None, :] // (side // p)
    ox = o[None, :] % (side // p)
    cy = (c[:, None] // side) // p
    cx = (c[:, None] % side) // p
    return ((cy == oy) & (cx == ox)).astype(F32) / float(p * p)


def _convt(x, w, b, faces, hw, ch):
    D = w.shape[1]
    y = _mm(x, w.reshape(ch, D * 4), jnp.repeat(b, 4))
    y = y.reshape(faces, hw, hw, D, 2, 2).transpose(0, 1, 4, 2, 5, 3)
    return y.reshape(faces * hw * hw * 4, D)


def kernel(in_feat, edge1, edge2, edge3, edge4, edge5, params,
           edge_index1, edge_index3, edge_index4, edge_index5):
    p = params
    p14 = _pool_mat(64, 4)
    p34 = _pool_mat(16, 2)
    p45 = _pool_mat(8, 2)

    x3 = _pool(in_feat.reshape(6, 4096, 7), p14).reshape(N3, 7)
    h2 = _level(p['mp1'], x3, edge3, edge_index3, N3, 32, E3)
    x4 = _pool(h2.reshape(6, 256, 32), p34).reshape(N4, 32)
    h3 = _level(p['mp2'], x4, edge4, edge_index4, N4, 64, E4)
    x5 = _pool(h3.reshape(6, 64, 64), p45).reshape(N5, 64)
    h4 = _level(p['mp3'], x5, edge5, edge_index5, N5, 128, E5)

    u1 = _convt(h4, p['up1_w'], p['up1_b'], 6, 4, 128)
    h6 = _level(p['mp4'], jnp.concatenate([u1, h3], axis=1),
                edge4, edge_index4, N4, 98, E4)
    u2 = _convt(h6, p['up2_w'], p['up2_b'], 6, 8, 98)
    h6 = _level(p['mp5'], jnp.concatenate([u2, h2], axis=1),
                edge3, edge_index3, N3, 60, E3)
    u3 = _convt(h6, p['up3_w'], p['up3_b'], 6, 16, 60)
    u4 = _convt(u3, p['up4_w'], p['up4_b'], 6, 32, 60)

    x1 = jnp.concatenate([u4, in_feat], axis=1)
    return _level1(p['mp6'], x1, edge1, edge_index1)


# speedup vs baseline: 1.8261x; 1.7092x over previous
import functools

import jax
import jax.numpy as jnp
from jax import lax
from jax.experimental import pallas as pl
from jax.experimental.pallas import tpu as pltpu
from jax.experimental.pallas import tpu_sc as plsc

F32 = jnp.float32

RES = 64
N1 = 6 * RES * RES
N3 = 6 * (RES // 4) ** 2
N4 = 6 * (RES // 8) ** 2
N5 = 6 * (RES // 16) ** 2
E1, E3, E4, E5 = 4 * N1, 4 * N3, 4 * N4, 4 * N5
EH = 32


def _mm_kernel(x_ref, w_ref, b_ref, o_ref):
    o_ref[...] = (jnp.dot(x_ref[...], w_ref[...], preferred_element_type=F32)
                  + b_ref[...])


def _mm(x, w, b):
    M, K = x.shape
    N = w.shape[1]
    return pl.pallas_call(
        _mm_kernel,
        out_shape=jax.ShapeDtypeStruct((M, N), F32),
    )(x, w, b.reshape(1, N))


def _pool_kernel(p_ref, x_ref, o_ref):
    o_ref[0] = lax.dot_general(p_ref[...], x_ref[0],
                               (((0,), (0,)), ((), ())),
                               preferred_element_type=F32)


def _pool(x, pmat):
    _, R, C = x.shape
    r = pmat.shape[1]
    return pl.pallas_call(
        _pool_kernel,
        grid=(6,),
        in_specs=[pl.BlockSpec((R, r), lambda i: (0, 0)),
                  pl.BlockSpec((1, R, C), lambda i: (i, 0, 0))],
        out_specs=pl.BlockSpec((1, r, C), lambda i: (i, 0, 0)),
        out_shape=jax.ShapeDtypeStruct((6, r, C), F32),
    )(pmat, x)


def _proj2_kernel(x_ref, w1_ref, b1_ref, w2_ref, b2_ref, o_ref):
    h = jnp.maximum(jnp.dot(x_ref[...], w1_ref[...],
                            preferred_element_type=F32) + b1_ref[...], 0.0)
    o_ref[...] = (jnp.dot(h, w2_ref[...], preferred_element_type=F32)
                  + b2_ref[...])


def _proj2(x, w1t, b1, w2t, b2, block_m=None):
    M, K = x.shape
    H = w1t.shape[1]
    O = w2t.shape[1]
    if block_m is None:
        block_m = M
    return pl.pallas_call(
        _proj2_kernel,
        grid=(M // block_m,),
        in_specs=[pl.BlockSpec((block_m, K), lambda i: (i, 0)),
                  pl.BlockSpec((K, H), lambda i: (0, 0)),
                  pl.BlockSpec((1, H), lambda i: (0, 0)),
                  pl.BlockSpec((H, O), lambda i: (0, 0)),
                  pl.BlockSpec((1, O), lambda i: (0, 0))],
        out_specs=pl.BlockSpec((block_m, O), lambda i: (i, 0)),
        out_shape=jax.ShapeDtypeStruct((M, O), F32),
    )(x, w1t, b1.reshape(1, H), w2t, b2.reshape(1, O))


def _level_kernel(x_ref, ef_ref, src_ref, dst_ref,
                  pn1t, pn1b, pn2t, pn2b, en1t, en1b, w3, bm, convb,
                  wiht, bih, whht, bhh, out_ref, ewh_ref, agg_ref,
                  *, N, E, H, CE):
    h = jnp.maximum(jnp.dot(x_ref[...], pn1t[...],
                            preferred_element_type=F32) + pn1b[...], 0.0)
    h = jnp.dot(h, pn2t[...], preferred_element_type=F32) + pn2b[...]
    hidden = h
    ewh_ref[...] = jnp.maximum(
        jnp.dot(ef_ref[...], en1t[...], preferred_element_type=F32)
        + en1b[...], 0.0)
    nch = E // CE

    def deg_body(c, deg):
        d = dst_ref[pl.ds(pl.multiple_of(c * CE, CE), CE), :]
        ohd = (lax.broadcasted_iota(jnp.int32, (CE, N), 1) == d).astype(F32)
        return deg + lax.dot_general(ohd, jnp.ones((CE, 1), F32),
                                     (((0,), (0,)), ((), ())),
                                     preferred_element_type=F32)

    deg = lax.fori_loop(0, nch, deg_body, jnp.zeros((N, 1), F32))
    dinv = 1.0 / jnp.maximum(deg, 1.0)

    for _ in range(3):
        node = hidden
        agg_ref[...] = jnp.zeros((N, H), F32)

        def chunk_body(c, _):
            o = pl.multiple_of(c * CE, CE)
            s = src_ref[pl.ds(o, CE), :]
            d = dst_ref[pl.ds(o, CE), :]
            iot = lax.broadcasted_iota(jnp.int32, (CE, N), 1)
            ohs = (iot == s).astype(F32)
            ohd = (iot == d).astype(F32)
            xs = jnp.dot(ohs, node, preferred_element_type=F32)
            ec = ewh_ref[pl.ds(o, CE), :]
            z = (xs[:, :, None] * ec[:, None, :]).reshape(CE, H * EH)
            msg = (jnp.dot(z, w3[...], preferred_element_type=F32)
                   + jnp.dot(xs, bm[...], preferred_element_type=F32))
            agg_ref[...] += lax.dot_general(ohd, msg, (((0,), (0,)), ((), ())),
                                            preferred_element_type=F32)
            return 0

        lax.fori_loop(0, nch, chunk_body, 0)
        m = jnp.maximum(agg_ref[...] * dinv + convb[...], 0.0)
        gi = jnp.dot(m, wiht[...], preferred_element_type=F32) + bih[...]
        gh = jnp.dot(hidden, whht[...], preferred_element_type=F32) + bhh[...]
        r = jax.nn.sigmoid(gi[:, :H] + gh[:, :H])
        zg = jax.nn.sigmoid(gi[:, H:2 * H] + gh[:, H:2 * H])
        n = jnp.tanh(gi[:, 2 * H:] + r * gh[:, 2 * H:])
        hidden = (1.0 - zg) * n + zg * hidden
    out_ref[...] = hidden


def _w3_bm(p, H):
    w3 = p['en2_w'].reshape(H, H, EH).transpose(0, 2, 1).reshape(H * EH, H)
    bm = p['en2_b'].reshape(H, H)
    return w3, bm


def _level(p, x, ef, ei, N, H, E):
    CE = min(E, 512)
    w3, bm = _w3_bm(p, H)
    kfn = functools.partial(_level_kernel, N=N, E=E, H=H, CE=CE)
    return pl.pallas_call(
        kfn,
        out_shape=jax.ShapeDtypeStruct((N, H), F32),
        scratch_shapes=[pltpu.VMEM((E, EH), F32),
                        pltpu.VMEM((N, H), F32)],
    )(x, ef, ei[0].reshape(E, 1), ei[1].reshape(E, 1),
      p['pn1_w'].T, p['pn1_b'].reshape(1, H),
      p['pn2_w'].T, p['pn2_b'].reshape(1, H),
      p['en1_w'].T, p['en1_b'].reshape(1, EH),
      w3, bm, p['conv_b'].reshape(1, H),
      p['gru_wih'].T, p['gru_bih'].reshape(1, 3 * H),
      p['gru_whh'].T, p['gru_bhh'].reshape(1, 3 * H))


_SC_MESH = plsc.VectorSubcoreMesh(core_axis_name="c", subcore_axis_name="s")
_NW = 32
_LW = 128


def _sc_gather(table, idx2d, E):
    bpw = E // _NW
    nch = bpw // 1024

    @functools.partial(
        pl.kernel,
        out_type=jax.ShapeDtypeStruct((E, _LW), F32),
        mesh=_SC_MESH,
        scratch_types=[pltpu.VMEM((8, 128), jnp.int32),
                       pltpu.VMEM((512, _LW), F32),
                       pltpu.SemaphoreType.DMA],
    )
    def k(table_hbm, idx_hbm, out_hbm, idx_v, rows_v, sem):
        wid = lax.axis_index("c") * 16 + lax.axis_index("s")
        base = wid * bpw

        @pl.loop(0, nch)
        def _chunk(ci):
            e0 = pl.multiple_of(base + ci * 1024, 1024)
            pltpu.sync_copy(
                idx_hbm.at[pl.ds(pl.multiple_of(e0 // 128, 8), 8)], idx_v)
            for h in range(2):
                hs = []
                for j in range(4):
                    hs.append(pltpu.async_copy(
                        table_hbm.at[idx_v.at[h * 4 + j]],
                        rows_v.at[pl.ds(j * 128, 128)], sem))
                for hdl in hs:
                    hdl.wait()
                pltpu.sync_copy(
                    rows_v,
                    out_hbm.at[pl.ds(pl.multiple_of(e0 + h * 512, 512), 512)])

    return k(table, idx2d)


def _sc_scatter_add(msg, dst2, zeros, N, E):
    half = N // 2
    bps = E // 16
    nch = bps // 1024

    @functools.partial(
        pl.kernel,
        out_type=jax.ShapeDtypeStruct((N, _LW), F32),
        mesh=_SC_MESH,
        scratch_types=[pltpu.VMEM((8, 128), jnp.int32),
                       pltpu.VMEM((128, _LW), F32),
                       pltpu.VMEM_SHARED((16 * 776, _LW), F32),
                       pltpu.SemaphoreType.DMA],
    )
    def k(msg_hbm, idx_hbm, z_hbm, out_hbm, idx_v, msg_v, acc, sem):
        c = lax.axis_index("c")
        s = lax.axis_index("s")
        pltpu.sync_copy(z_hbm, acc.at[pl.ds(pl.multiple_of(s * 776, 8), 776)])
        plsc.subcore_barrier()
        base = s * bps

        @pl.loop(0, nch)
        def _chunk(ci):
            e0 = pl.multiple_of(base + ci * 1024, 1024)
            pltpu.sync_copy(
                idx_hbm.at[c].at[pl.ds(pl.multiple_of(e0 // 128, 8), 8)],
                idx_v)
            for h in range(8):
                pltpu.sync_copy(
                    msg_hbm.at[pl.ds(pl.multiple_of(e0 + h * 128, 128), 128)],
                    msg_v)
                pltpu.async_copy(msg_v, acc.at[idx_v.at[h]], sem,
                                 add=True).wait()
        plsc.subcore_barrier()
        r0 = pl.multiple_of(s * (half // 16), 8)
        w0 = pl.multiple_of(c * half + s * (half // 16), 8)
        pltpu.sync_copy(acc.at[pl.ds(r0, half // 16)],
                        out_hbm.at[pl.ds(w0, half // 16)])

    return k(msg, dst2, zeros)


def _msg_kernel(xs_ref, sm_ref, ef_ref, en1rt, en1rb, wa, bm, o_ref,
                *, H, with_ones):
    x128 = xs_ref[...]
    sm = sm_ref[...]
    EB = x128.shape[0]
    xs = jnp.zeros((EB, H), F32)
    for kk in range(4):
        sel = (sm == kk).astype(F32)
        xs = xs + sel * x128[:, kk * H:(kk + 1) * H]
    a = jnp.dot(xs, wa[...], preferred_element_type=F32)
    er = jnp.maximum(jnp.dot(ef_ref[...], en1rt[...],
                             preferred_element_type=F32) + en1rb[...], 0.0)
    p = a * er
    w = H * EH
    while w > H:
        w //= 2
        p = p[:, :w] + p[:, w:2 * w]
    msg = p + jnp.dot(xs, bm[...], preferred_element_type=F32)
    deg = jnp.ones if with_ones else jnp.zeros
    o_ref[...] = jnp.concatenate(
        [msg, deg((EB, 16), F32), jnp.zeros((EB, 128 - H - 16), F32)], axis=1)


def _msg(xs, sm, ef, en1rt, en1rb, wa, bm, H, with_ones):
    E = xs.shape[0]
    EB = 2048
    kfn = functools.partial(_msg_kernel, H=H, with_ones=with_ones)
    return pl.pallas_call(
        kfn,
        grid=(E // EB,),
        in_specs=[pl.BlockSpec((EB, 128), lambda i: (i, 0)),
                  pl.BlockSpec((EB, 1), lambda i: (i, 0)),
                  pl.BlockSpec((EB, 4), lambda i: (i, 0)),
                  pl.BlockSpec((4, EH * H), lambda i: (0, 0)),
                  pl.BlockSpec((1, EH * H), lambda i: (0, 0)),
                  pl.BlockSpec((H, EH * H), lambda i: (0, 0)),
                  pl.BlockSpec((H, H), lambda i: (0, 0))],
        out_specs=pl.BlockSpec((EB, 128), lambda i: (i, 0)),
        out_shape=jax.ShapeDtypeStruct((E, 128), F32),
    )(xs, sm, ef, en1rt, en1rb.reshape(1, EH * H), wa, bm)


def _gru1_kernel(agg_ref, hid_ref, convb, wiht, bih, whht, bhh,
                 out_ref, dinv_ref, *, H):
    agg = agg_ref[:, :H]
    deg = agg_ref[:, H:H + 1]
    dinv = 1.0 / jnp.maximum(deg, 1.0)
    dinv_ref[...] = dinv
    _gru_core(agg, dinv, hid_ref, convb, wiht, bih, whht, bhh, out_ref, H)


def _gru23_kernel(agg_ref, dinv_ref, hid_ref, convb, wiht, bih, whht, bhh,
                  out_ref, *, H):
    _gru_core(agg_ref[:, :H], dinv_ref[...], hid_ref, convb, wiht, bih,
              whht, bhh, out_ref, H)


def _gru_core(agg, dinv, hid_ref, convb, wiht, bih, whht, bhh, out_ref, H):
    hidden = hid_ref[...]
    m = jnp.maximum(agg * dinv + convb[...], 0.0)
    gi = jnp.dot(m, wiht[...], preferred_element_type=F32) + bih[...]
    gh = jnp.dot(hidden, whht[...], preferred_element_type=F32) + bhh[...]
    r = jax.nn.sigmoid(gi[:, :H] + gh[:, :H])
    zg = jax.nn.sigmoid(gi[:, H:2 * H] + gh[:, H:2 * H])
    n = jnp.tanh(gi[:, 2 * H:] + r * gh[:, 2 * H:])
    out_ref[...] = (1.0 - zg) * n + zg * hidden


def _gru1(agg, hidden, p, N, H):
    NB = 8192
    kfn = functools.partial(_gru1_kernel, H=H)
    return pl.pallas_call(
        kfn,
        grid=(N // NB,),
        in_specs=[pl.BlockSpec((NB, 128), lambda i: (i, 0)),
                  pl.BlockSpec((NB, H), lambda i: (i, 0)),
                  pl.BlockSpec((1, H), lambda i: (0, 0)),
                  pl.BlockSpec((H, 3 * H), lambda i: (0, 0)),
                  pl.BlockSpec((1, 3 * H), lambda i: (0, 0)),
                  pl.BlockSpec((H, 3 * H), lambda i: (0, 0)),
                  pl.BlockSpec((1, 3 * H), lambda i: (0, 0))],
        out_specs=[pl.BlockSpec((NB, H), lambda i: (i, 0)),
                   pl.BlockSpec((NB, 1), lambda i: (i, 0))],
        out_shape=[jax.ShapeDtypeStruct((N, H), F32),
                   jax.ShapeDtypeStruct((N, 1), F32)],
    )(agg, hidden, p['conv_b'].reshape(1, H),
      p['gru_wih'].T, p['gru_bih'].reshape(1, 3 * H),
      p['gru_whh'].T, p['gru_bhh'].reshape(1, 3 * H))


def _gru23(agg, dinv, hidden, p, N, H):
    NB = 8192
    kfn = functools.partial(_gru23_kernel, H=H)
    return pl.pallas_call(
        kfn,
        grid=(N // NB,),
        in_specs=[pl.BlockSpec((NB, 128), lambda i: (i, 0)),
                  pl.BlockSpec((NB, 1), lambda i: (i, 0)),
                  pl.BlockSpec((NB, H), lambda i: (i, 0)),
                  pl.BlockSpec((1, H), lambda i: (0, 0)),
                  pl.BlockSpec((H, 3 * H), lambda i: (0, 0)),
                  pl.BlockSpec((1, 3 * H), lambda i: (0, 0)),
                  pl.BlockSpec((H, 3 * H), lambda i: (0, 0)),
                  pl.BlockSpec((1, 3 * H), lambda i: (0, 0))],
        out_specs=pl.BlockSpec((NB, H), lambda i: (i, 0)),
        out_shape=jax.ShapeDtypeStruct((N, H), F32),
    )(agg, dinv, hidden, p['conv_b'].reshape(1, H),
      p['gru_wih'].T, p['gru_bih'].reshape(1, 3 * H),
      p['gru_whh'].T, p['gru_bhh'].reshape(1, 3 * H))


def _level1(p, x, ef, ei):
    H = 32
    N, E = N1, E1
    half = N // 2
    w3, bm = _w3_bm(p, H)
    wa = w3.reshape(H, EH * H)
    en1rt = jnp.repeat(p['en1_w'].T, H, axis=1)
    en1rb = jnp.repeat(p['en1_b'], H)
    src, dst = ei[0], ei[1]
    srcdiv = (src // 4).reshape(E // 128, 128)
    srcmod = (src % 4).reshape(E, 1)
    dst_lo = jnp.where(dst < half, dst, half)
    dst_hi = jnp.where(dst >= half, dst - half, half)
    dst2 = jnp.stack([dst_lo, dst_hi]).reshape(2, E // 128, 128)
    zeros = jnp.zeros((776, 128), F32)

    hidden = _proj2(x, p['pn1_w'].T, p['pn1_b'], p['pn2_w'].T, p['pn2_b'],
                    block_m=4096)
    dinv = None
    for step in range(3):
        xs = _sc_gather(hidden.reshape(N // 4, 128), srcdiv, E)
        first = step == 0
        msg = _msg(xs, srcmod, ef, en1rt, en1rb, wa, bm, H,
                   with_ones=first)
        agg = _sc_scatter_add(msg, dst2, zeros, N, E)
        if first:
            hidden, dinv = _gru1(agg, hidden, p, N, H)
        else:
            hidden = _gru23(agg, dinv, hidden, p, N, H)
    return _proj2(hidden, p['dec1_w'].T, p['dec1_b'],
                  p['dec2_w'].T, p['dec2_b'], block_m=4096)


def _pool_mat(side, p):
    c = jnp.arange(side * side)
    o = jnp.arange((side // p) ** 2)
    oy = o[None, :] // (side // p)
    ox = o[None, :] % (side // p)
    cy = (c[:, None] // side) // p
    cx = (c[:, None] % side) // p
    return ((cy == oy) & (cx == ox)).astype(F32) / float(p * p)


def _convt(x, w, b, faces, hw, ch):
    D = w.shape[1]
    y = _mm(x, w.reshape(ch, D * 4), jnp.repeat(b, 4))
    y = y.reshape(faces, hw, hw, D, 2, 2).transpose(0, 1, 4, 2, 5, 3)
    return y.reshape(faces * hw * hw * 4, D)


def kernel(in_feat, edge1, edge2, edge3, edge4, edge5, params,
           edge_index1, edge_index3, edge_index4, edge_index5):
    p = params
    p14 = _pool_mat(64, 4)
    p34 = _pool_mat(16, 2)
    p45 = _pool_mat(8, 2)

    x3 = _pool(in_feat.reshape(6, 4096, 7), p14).reshape(N3, 7)
    h2 = _level(p['mp1'], x3, edge3, edge_index3, N3, 32, E3)
    x4 = _pool(h2.reshape(6, 256, 32), p34).reshape(N4, 32)
    h3 = _level(p['mp2'], x4, edge4, edge_index4, N4, 64, E4)
    x5 = _pool(h3.reshape(6, 64, 64), p45).reshape(N5, 64)
    h4 = _level(p['mp3'], x5, edge5, edge_index5, N5, 128, E5)

    u1 = _convt(h4, p['up1_w'], p['up1_b'], 6, 4, 128)
    h6 = _level(p['mp4'], jnp.concatenate([u1, h3], axis=1),
                edge4, edge_index4, N4, 98, E4)
    u2 = _convt(h6, p['up2_w'], p['up2_b'], 6, 8, 98)
    h6 = _level(p['mp5'], jnp.concatenate([u2, h2], axis=1),
                edge3, edge_index3, N3, 60, E3)
    u3 = _convt(h6, p['up3_w'], p['up3_b'], 6, 16, 60)
    u4 = _convt(u3, p['up4_w'], p['up4_b'], 6, 32, 60)

    x1 = jnp.concatenate([u4, in_feat], axis=1)
    return _level1(p['mp6'], x1, edge1, edge_index1)
